# baseline (speedup 1.0000x reference)
import jax
import jax.numpy as jnp
from jax import lax
from jax.experimental import pallas as pl
from jax.experimental.pallas import tpu as pltpu
from jax.experimental.pallas import tpu_sc as plsc

HID = 768
B = 4
S = 2048
EPS = 1e-12

N = B * S
NC = 2
NS = 16
NW = NC * NS
ROWS_PER_W = N // NW
CHUNK = 16
NCHUNK = ROWS_PER_W // CHUNK
NBUF = 4


def _gather_body(ids_hbm, table_hbm, out_hbm, idx_v, b0, b1, b2, b3,
                 g0, g1, g2, g3, w0, w1, w2, w3):
    wid = lax.axis_index("s") * NC + lax.axis_index("c")
    base = wid * ROWS_PER_W
    pltpu.sync_copy(ids_hbm.at[wid], idx_v)
    bufs = (b0, b1, b2, b3)
    gsems = (g0, g1, g2, g3)
    wsems = (w0, w1, w2, w3)
    gcps = {}
    wcps = {}
    for c in range(NBUF - 1):
        gcps[c] = pltpu.async_copy(
            table_hbm.at[idx_v.at[c]], bufs[c], gsems[c])
    for c in range(NCHUNK):
        gcps[c].wait()
        wcps[c] = pltpu.async_copy(
            bufs[c % NBUF],
            out_hbm.at[pl.ds(base + c * CHUNK, CHUNK)],
            wsems[c % NBUF],
        )
        if c + NBUF - 1 < NCHUNK:
            if c >= 1:
                wcps[c - 1].wait()
            gcps[c + NBUF - 1] = pltpu.async_copy(
                table_hbm.at[idx_v.at[c + NBUF - 1]],
                bufs[(c + NBUF - 1) % NBUF],
                gsems[(c + NBUF - 1) % NBUF],
            )
    for c in range(NCHUNK - NBUF, NCHUNK):
        if c >= 0:
            wcps[c].wait()


def _sc_gather(ids3, word_emb):
    mesh = plsc.VectorSubcoreMesh(core_axis_name="c", subcore_axis_name="s")
    run = pl.kernel(
        _gather_body,
        mesh=mesh,
        out_type=jax.ShapeDtypeStruct((N, HID), jnp.float32),
        scratch_types=[
            pltpu.VMEM((NCHUNK, CHUNK), jnp.int32),
            pltpu.VMEM((CHUNK, HID), jnp.float32),
            pltpu.VMEM((CHUNK, HID), jnp.float32),
            pltpu.VMEM((CHUNK, HID), jnp.float32),
            pltpu.VMEM((CHUNK, HID), jnp.float32),
            pltpu.SemaphoreType.DMA,
            pltpu.SemaphoreType.DMA,
            pltpu.SemaphoreType.DMA,
            pltpu.SemaphoreType.DMA,
            pltpu.SemaphoreType.DMA,
            pltpu.SemaphoreType.DMA,
            pltpu.SemaphoreType.DMA,
            pltpu.SemaphoreType.DMA,
        ],
    )
    return run(ids3, word_emb)


def _ln_body(tt_ref, x_ref, pos_ref, type_ref, gamma_ref, beta_ref, out_ref):
    x = x_ref[...] + pos_ref[...]
    tt = tt_ref[0, 0, :].astype(jnp.float32)[:, None]
    t0 = type_ref[0:1, :]
    t1 = type_ref[1:2, :]
    x = x + t0 + tt * (t1 - t0)
    mean = jnp.mean(x, axis=1, keepdims=True)
    xc = x - mean
    var = jnp.mean(xc * xc, axis=1, keepdims=True)
    inv = lax.rsqrt(var + EPS)
    out_ref[...] = xc * inv * gamma_ref[...] + beta_ref[...]


def _tc_layernorm(tt3, gathered, pos_emb, type_emb, gamma2, beta2):
    return pl.pallas_call(
        _ln_body,
        grid=(B,),
        in_specs=[
            pl.BlockSpec((1, 1, S), lambda i: (i, 0, 0)),
            pl.BlockSpec((S, HID), lambda i: (i, 0)),
            pl.BlockSpec((S, HID), lambda i: (0, 0)),
            pl.BlockSpec((2, HID), lambda i: (0, 0)),
            pl.BlockSpec((1, HID), lambda i: (0, 0)),
            pl.BlockSpec((1, HID), lambda i: (0, 0)),
        ],
        out_specs=pl.BlockSpec((S, HID), lambda i: (i, 0)),
        out_shape=jax.ShapeDtypeStruct((N, HID), jnp.float32),
    )(tt3, gathered, pos_emb, type_emb, gamma2, beta2)


def kernel(input_ids, token_type_ids, word_emb, pos_emb, type_emb, gamma, beta):
    ids3 = input_ids.reshape(NW, NCHUNK, CHUNK).astype(jnp.int32)
    gathered = _sc_gather(ids3, word_emb)
    tt3 = token_type_ids.reshape(B, 1, S).astype(jnp.int32)
    out = _tc_layernorm(
        tt3,
        gathered,
        pos_emb,
        type_emb,
        gamma.reshape(1, HID),
        beta.reshape(1, HID),
    )
    return out.reshape(B, S, HID)


# speedup vs baseline: 1.0152x; 1.0152x over previous
import jax
import jax.numpy as jnp
from jax import lax
from jax.experimental import pallas as pl
from jax.experimental.pallas import tpu as pltpu
from jax.experimental.pallas import tpu_sc as plsc

HID = 768
B = 4
S = 2048
EPS = 1e-12

N = B * S
NC = 2
NS = 16
NW = NC * NS
ROWS_PER_W = N // NW
CHUNK = 32
NCHUNK = ROWS_PER_W // CHUNK
NBUF = 4


def _gather_body(ids_hbm, table_hbm, out_hbm, idx_v, b0, b1, b2, b3,
                 g0, g1, g2, g3, w0, w1, w2, w3):
    wid = lax.axis_index("s") * NC + lax.axis_index("c")
    base = wid * ROWS_PER_W
    pltpu.sync_copy(ids_hbm.at[wid], idx_v)
    bufs = (b0, b1, b2, b3)
    gsems = (g0, g1, g2, g3)
    wsems = (w0, w1, w2, w3)
    gcps = {}
    wcps = {}
    for c in range(NBUF - 1):
        gcps[c] = pltpu.async_copy(
            table_hbm.at[idx_v.at[c]], bufs[c], gsems[c])
    for c in range(NCHUNK):
        gcps[c].wait()
        wcps[c] = pltpu.async_copy(
            bufs[c % NBUF],
            out_hbm.at[pl.ds(base + c * CHUNK, CHUNK)],
            wsems[c % NBUF],
        )
        if c + NBUF - 1 < NCHUNK:
            if c >= 1:
                wcps[c - 1].wait()
            gcps[c + NBUF - 1] = pltpu.async_copy(
                table_hbm.at[idx_v.at[c + NBUF - 1]],
                bufs[(c + NBUF - 1) % NBUF],
                gsems[(c + NBUF - 1) % NBUF],
            )
    for c in range(NCHUNK - NBUF, NCHUNK):
        if c >= 0:
            wcps[c].wait()


def _sc_gather(ids3, word_emb):
    mesh = plsc.VectorSubcoreMesh(core_axis_name="c", subcore_axis_name="s")
    run = pl.kernel(
        _gather_body,
        mesh=mesh,
        out_type=jax.ShapeDtypeStruct((N, HID), jnp.float32),
        scratch_types=[
            pltpu.VMEM((NCHUNK, CHUNK), jnp.int32),
            pltpu.VMEM((CHUNK, HID), jnp.float32),
            pltpu.VMEM((CHUNK, HID), jnp.float32),
            pltpu.VMEM((CHUNK, HID), jnp.float32),
            pltpu.VMEM((CHUNK, HID), jnp.float32),
            pltpu.SemaphoreType.DMA,
            pltpu.SemaphoreType.DMA,
            pltpu.SemaphoreType.DMA,
            pltpu.SemaphoreType.DMA,
            pltpu.SemaphoreType.DMA,
            pltpu.SemaphoreType.DMA,
            pltpu.SemaphoreType.DMA,
            pltpu.SemaphoreType.DMA,
        ],
    )
    return run(ids3, word_emb)


def _ln_body(tt_ref, x_ref, pos_ref, type_ref, gamma_ref, beta_ref, out_ref):
    x = x_ref[...] + pos_ref[...]
    tt = tt_ref[0, 0, :].astype(jnp.float32)[:, None]
    t0 = type_ref[0:1, :]
    t1 = type_ref[1:2, :]
    x = x + t0 + tt * (t1 - t0)
    mean = jnp.mean(x, axis=1, keepdims=True)
    xc = x - mean
    var = jnp.mean(xc * xc, axis=1, keepdims=True)
    inv = lax.rsqrt(var + EPS)
    out_ref[...] = xc * inv * gamma_ref[...] + beta_ref[...]


def _tc_layernorm(tt3, gathered, pos_emb, type_emb, gamma2, beta2):
    return pl.pallas_call(
        _ln_body,
        grid=(B,),
        in_specs=[
            pl.BlockSpec((1, 1, S), lambda i: (i, 0, 0)),
            pl.BlockSpec((S, HID), lambda i: (i, 0)),
            pl.BlockSpec((S, HID), lambda i: (0, 0)),
            pl.BlockSpec((2, HID), lambda i: (0, 0)),
            pl.BlockSpec((1, HID), lambda i: (0, 0)),
            pl.BlockSpec((1, HID), lambda i: (0, 0)),
        ],
        out_specs=pl.BlockSpec((S, HID), lambda i: (i, 0)),
        out_shape=jax.ShapeDtypeStruct((N, HID), jnp.float32),
    )(tt3, gathered, pos_emb, type_emb, gamma2, beta2)


def kernel(input_ids, token_type_ids, word_emb, pos_emb, type_emb, gamma, beta):
    ids3 = input_ids.reshape(NW, NCHUNK, CHUNK).astype(jnp.int32)
    gathered = _sc_gather(ids3, word_emb)
    tt3 = token_type_ids.reshape(B, 1, S).astype(jnp.int32)
    out = _tc_layernorm(
        tt3,
        gathered,
        pos_emb,
        type_emb,
        gamma.reshape(1, HID),
        beta.reshape(1, HID),
    )
    return out.reshape(B, S, HID)
